# speedup vs baseline: 32.3705x; 32.3705x over previous
import jax
import jax.numpy as jnp
from jax import lax
from jax.experimental import pallas as pl
from jax.experimental.pallas import tpu as pltpu

_T = 168
_NT = 25
_WIND_W = 4
_WIND_IDX = 10
_WSPM_MEAN = 2.5
_WSPM_SCALE = 1.8
_MAX_DELAY = 24.0


def _tc_body(xs_ref, adj_ref, dist_ref, out_ref):
    wind = xs_ref[0, _NT - _WIND_W:, :, _WIND_IDX]
    wspm = jnp.clip(jnp.mean(wind, axis=0) * _WSPM_SCALE + _WSPM_MEAN, 0.0, None)
    speed = wspm * 3.6 + 0.001
    tau = jnp.clip(dist_ref[...] / speed[None, :], 0.0, _MAX_DELAY)
    tq = (_NT - 1.0) - tau
    adj = adj_ref[0]
    acc = jnp.zeros((128, 32), dtype=jnp.float32)
    for t in range(_NT):
        w = adj * jnp.maximum(0.0, 1.0 - jnp.abs(t - tq))
        acc = acc + jnp.dot(w, xs_ref[0, t], preferred_element_type=jnp.float32)
    out_ref[0] = acc


def kernel(x_raw, adj, dist_km):
    B, T, N, F = x_raw.shape
    xs = lax.slice_in_dim(x_raw, T - _NT, T, axis=1)
    return pl.pallas_call(
        _tc_body,
        grid=(B,),
        in_specs=[
            pl.BlockSpec((1, _NT, N, F), lambda b: (b, 0, 0, 0)),
            pl.BlockSpec((1, N, N), lambda b: (b, 0, 0)),
            pl.BlockSpec((N, N), lambda b: (0, 0)),
        ],
        out_specs=pl.BlockSpec((1, N, F), lambda b: (b, 0, 0)),
        out_shape=jax.ShapeDtypeStruct((B, N, F), jnp.float32),
    )(xs, adj, dist_km)
